# Initial kernel scaffold; baseline (speedup 1.0000x reference)
import jax
import jax.numpy as jnp
from jax.experimental import pallas as pl


def kernel(x, edge_index, params):
    raise NotImplementedError("write your pallas kernel here")


# speedup vs baseline: 4.5146x; 4.5146x over previous
import functools

import jax
import jax.numpy as jnp
from jax import lax
from jax.experimental import pallas as pl
from jax.experimental.pallas import tpu as pltpu
from jax.experimental.pallas import tpu_sc as plsc

N = 10000
E = 320000
D = 128
H = 16

NCORES = 2
NSUB = 16
NW = NCORES * NSUB
EPT = E // NW
CH = 80
NCHUNK = EPT // CH
NPAD = 10240
NPT = NPAD // NSUB

ROW_BLK = 1000


_mesh = plsc.VectorSubcoreMesh(core_axis_name="c", subcore_axis_name="s")


@functools.partial(
    pl.kernel,
    out_type=jax.ShapeDtypeStruct((NCORES, NPAD, D), jnp.float32),
    mesh=_mesh,
    scratch_types=[
        pltpu.VMEM((CH,), jnp.int32),
        pltpu.VMEM((CH,), jnp.int32),
        pltpu.VMEM((CH, D), jnp.float32),
        pltpu.VMEM_SHARED((NPAD, D), jnp.float32),
        pltpu.SemaphoreType.DMA,
    ],
)
def _edge_aggr(m_hbm, src_hbm, dst_hbm, zeros_hbm, out_hbm,
               src_v, dst_v, rows_v, aggr_sh, sem):
    c = lax.axis_index("c")
    s = lax.axis_index("s")
    eb = (c * NSUB + s) * EPT

    pltpu.sync_copy(zeros_hbm, aggr_sh.at[pl.ds(s * NPT, NPT)])
    plsc.subcore_barrier()

    def body(t, carry):
        off = eb + t * CH
        pltpu.sync_copy(src_hbm.at[pl.ds(off, CH)], src_v)
        pltpu.sync_copy(dst_hbm.at[pl.ds(off, CH)], dst_v)
        pltpu.async_copy(m_hbm.at[src_v], rows_v, sem).wait()
        pltpu.sync_copy(rows_v, aggr_sh.at[dst_v], add=True)
        return carry

    lax.fori_loop(0, NCHUNK, body, 0)

    plsc.subcore_barrier()
    pltpu.sync_copy(aggr_sh.at[pl.ds(s * NPT, NPT)],
                    out_hbm.at[c, pl.ds(s * NPT, NPT)])


def _msg_body(x_ref, w1, b1, w2, b2, o_ref):
    h = jnp.maximum(
        jnp.dot(x_ref[...], w1[...], preferred_element_type=jnp.float32)
        + b1[...], 0.0)
    o_ref[...] = jnp.maximum(
        jnp.dot(h, w2[...], preferred_element_type=jnp.float32) + b2[...], 0.0)


def _full(shape):
    return pl.BlockSpec(shape, lambda i: (0, 0))


def _node_msg(x, w1t, b1, w2t, b2):
    return pl.pallas_call(
        _msg_body,
        grid=(N // ROW_BLK,),
        in_specs=[
            pl.BlockSpec((ROW_BLK, D), lambda i: (i, 0)),
            _full((D, H)), _full((1, H)), _full((H, D)), _full((1, D)),
        ],
        out_specs=pl.BlockSpec((ROW_BLK, D), lambda i: (i, 0)),
        out_shape=jax.ShapeDtypeStruct((N, D), jnp.float32),
    )(x, w1t, b1, w2t, b2)


def _upd_body(a0, a1, x_ref, u1a, u1b, ub1, u2t, ub2, o_ref):
    aggr = a0[...] + a1[...]
    h = jnp.maximum(
        jnp.dot(aggr, u1a[...], preferred_element_type=jnp.float32)
        + jnp.dot(x_ref[...], u1b[...], preferred_element_type=jnp.float32)
        + ub1[...], 0.0)
    o_ref[...] = jnp.maximum(
        jnp.dot(h, u2t[...], preferred_element_type=jnp.float32) + ub2[...],
        0.0)


def _upd_msg_body(a0, a1, x_ref, u1a, u1b, ub1, u2t, ub2,
                  m1t, mb1, m2t, mb2, nx_ref, m_ref):
    _upd_body(a0, a1, x_ref, u1a, u1b, ub1, u2t, ub2, nx_ref)
    _msg_body(nx_ref, m1t, mb1, m2t, mb2, m_ref)


def _update(a0, a1, x, u1a, u1b, ub1, u2t, ub2):
    blk = pl.BlockSpec((ROW_BLK, D), lambda i: (i, 0))
    return pl.pallas_call(
        _upd_body,
        grid=(N // ROW_BLK,),
        in_specs=[blk, blk, blk,
                  _full((D, H)), _full((D, H)), _full((1, H)),
                  _full((H, D)), _full((1, D))],
        out_specs=blk,
        out_shape=jax.ShapeDtypeStruct((N, D), jnp.float32),
    )(a0, a1, x, u1a, u1b, ub1, u2t, ub2)


def _update_msg(a0, a1, x, u1a, u1b, ub1, u2t, ub2, m1t, mb1, m2t, mb2):
    blk = pl.BlockSpec((ROW_BLK, D), lambda i: (i, 0))
    return pl.pallas_call(
        _upd_msg_body,
        grid=(N // ROW_BLK,),
        in_specs=[blk, blk, blk,
                  _full((D, H)), _full((D, H)), _full((1, H)),
                  _full((H, D)), _full((1, D)),
                  _full((D, H)), _full((1, H)), _full((H, D)), _full((1, D))],
        out_specs=[blk, blk],
        out_shape=[jax.ShapeDtypeStruct((N, D), jnp.float32),
                   jax.ShapeDtypeStruct((N, D), jnp.float32)],
    )(a0, a1, x, u1a, u1b, ub1, u2t, ub2, m1t, mb1, m2t, mb2)


def _prep_mlp(p):
    return (p['W1'].T, p['b1'].reshape(1, -1), p['W2'].T,
            p['b2'].reshape(1, -1))


def kernel(x, edge_index, params):
    src = edge_index[0].astype(jnp.int32)
    dst = edge_index[1].astype(jnp.int32)
    zeros = jnp.zeros((NPT, D), jnp.float32)

    msg_w = [_prep_mlp(p['mlp']) for p in params]
    upd_w = []
    for p in params:
        u1t = p['update']['W1'].T
        upd_w.append((u1t[:D], u1t[D:], p['update']['b1'].reshape(1, -1),
                      p['update']['W2'].T, p['update']['b2'].reshape(1, -1)))

    m = _node_msg(x, *msg_w[0])
    for l in range(3):
        partials = _edge_aggr(m, src, dst, zeros)
        a0, a1 = partials[0, :N], partials[1, :N]
        if l < 2:
            x, m = _update_msg(a0, a1, x, *upd_w[l], *msg_w[l + 1])
        else:
            x = _update(a0, a1, x, *upd_w[l])
    return x
